# baseline (speedup 1.0000x reference)
import functools

import jax
import jax.numpy as jnp
from jax.experimental import pallas as pl
from jax.experimental.pallas import tpu as pltpu

MAX_DIST = 32
TABLE_SIZE = 2 * MAX_DIST + 1
VALS = MAX_DIST + 1
K = 128
NSLOT = 2


def _noop_kernel(out_ref):
    pass


def _bias_kernel(o_all_ref, m_ref, buf_ref, out_ref, scr, sem,
                 *, nh, nsteps):
    b = pl.program_id(0)
    g = pl.program_id(1)
    step = b * (nh // 2) + g
    slot = jax.lax.rem(step, NSLOT)
    hhi = g + nh // 2

    of = o_all_ref[0]

    def one_head(mm):
        a = jnp.dot(of, mm.astype(jnp.bfloat16),
                    preferred_element_type=jnp.float32)
        return jax.lax.dot_general(
            a.astype(jnp.bfloat16), of, (((1,), (1,)), ((), ())),
            preferred_element_type=jnp.float32)

    out_ref[0, 0] = one_head(m_ref[0, 0])

    @pl.when(step >= NSLOT)
    def _wait_prev():
        pltpu.make_async_copy(
            scr.at[slot], buf_ref.at[b, hhi], sem.at[slot]).wait()

    scr[slot] = one_head(m_ref[0, 1])
    pltpu.make_async_copy(
        scr.at[slot], buf_ref.at[b, hhi], sem.at[slot]).start()

    @pl.when(step == nsteps - 1)
    def _drain():
        for k in range(NSLOT):
            so = nsteps - NSLOT + k
            sl = so % NSLOT
            bo = so // (nh // 2)
            go = so % (nh // 2)
            pltpu.make_async_copy(
                scr.at[sl], buf_ref.at[bo, go + nh // 2],
                sem.at[sl]).wait()


@functools.partial(jax.jit, static_argnames=())
def kernel(positions, rel_bias_d, rel_bias_h, rel_bias_w):
    B, N, _ = positions.shape
    H = rel_bias_d.shape[1]

    pos = jnp.clip(positions.astype(jnp.int32), 0, MAX_DIST)
    ks = jnp.arange(K, dtype=jnp.int32)
    onehot = ((pos[:, :, 0, None] == ks)
              | (pos[:, :, 1, None] + VALS == ks)
              | (pos[:, :, 2, None] + 2 * VALS == ks)).astype(jnp.bfloat16)

    u = jnp.arange(VALS, dtype=jnp.int32)
    duv = u[:, None] - u[None, :] + MAX_DIST
    md = rel_bias_d[duv].transpose(2, 0, 1)
    mh = rel_bias_h[duv].transpose(2, 0, 1)
    mw = rel_bias_w[duv].transpose(2, 0, 1)
    m = jnp.zeros((H, K, K), dtype=jnp.float32)
    m = m.at[:, 0:VALS, 0:VALS].set(md)
    m = m.at[:, VALS:2 * VALS, VALS:2 * VALS].set(mh)
    m = m.at[:, 2 * VALS:3 * VALS, 2 * VALS:3 * VALS].set(mw)

    m_pairs = jnp.stack([m[: H // 2], m[H // 2:]], axis=1)

    buf = pl.pallas_call(
        _noop_kernel,
        out_specs=pl.BlockSpec(memory_space=pl.ANY),
        out_shape=jax.ShapeDtypeStruct((B, H, N, N), jnp.float32),
    )()

    grid = (B, H // 2)
    out = pl.pallas_call(
        functools.partial(_bias_kernel, nh=H, nsteps=B * (H // 2)),
        grid=grid,
        in_specs=[
            pl.BlockSpec((1, N, K), lambda b, g: (b, 0, 0)),
            pl.BlockSpec((1, 2, K, K), lambda b, g: (g, 0, 0, 0)),
            pl.BlockSpec(memory_space=pl.ANY),
        ],
        out_specs=pl.BlockSpec((1, 1, N, N), lambda b, g: (b, g, 0, 0)),
        out_shape=jax.ShapeDtypeStruct((B, H, N, N), jnp.float32),
        input_output_aliases={2: 0},
        scratch_shapes=[
            pltpu.VMEM((NSLOT, N, N), jnp.float32),
            pltpu.SemaphoreType.DMA((NSLOT,)),
        ],
    )(onehot, m_pairs, buf)
    return out


# speedup vs baseline: 1.0031x; 1.0031x over previous
import functools

import jax
import jax.numpy as jnp
from jax.experimental import pallas as pl

MAX_DIST = 32
TABLE_SIZE = 2 * MAX_DIST + 1
VALS = MAX_DIST + 1
K = 128


def _bias_kernel(o_all_ref, m_ref, out_ref, *, hb):
    of = o_all_ref[0]
    for hh in range(hb):
        m = m_ref[hh].astype(jnp.bfloat16)
        a = jnp.dot(of, m, preferred_element_type=jnp.float32)
        out = jax.lax.dot_general(
            a.astype(jnp.bfloat16), of, (((1,), (1,)), ((), ())),
            preferred_element_type=jnp.float32)
        out_ref[0, hh] = out


@functools.partial(jax.jit, static_argnames=())
def kernel(positions, rel_bias_d, rel_bias_h, rel_bias_w):
    B, N, _ = positions.shape
    H = rel_bias_d.shape[1]
    HB = 2

    pos = jnp.clip(positions.astype(jnp.int32), 0, MAX_DIST)
    ks = jnp.arange(K, dtype=jnp.int32)
    onehot = ((pos[:, :, 0, None] == ks)
              | (pos[:, :, 1, None] + VALS == ks)
              | (pos[:, :, 2, None] + 2 * VALS == ks)).astype(jnp.bfloat16)

    u = jnp.arange(VALS, dtype=jnp.int32)
    duv = u[:, None] - u[None, :] + MAX_DIST
    md = rel_bias_d[duv].transpose(2, 0, 1)
    mh = rel_bias_h[duv].transpose(2, 0, 1)
    mw = rel_bias_w[duv].transpose(2, 0, 1)
    m = jnp.zeros((H, K, K), dtype=jnp.float32)
    m = m.at[:, 0:VALS, 0:VALS].set(md)
    m = m.at[:, VALS:2 * VALS, VALS:2 * VALS].set(mh)
    m = m.at[:, 2 * VALS:3 * VALS, 2 * VALS:3 * VALS].set(mw)

    grid = (B, H // HB)
    out = pl.pallas_call(
        functools.partial(_bias_kernel, hb=HB),
        grid=grid,
        in_specs=[
            pl.BlockSpec((1, N, K), lambda b, hg: (b, 0, 0)),
            pl.BlockSpec((HB, K, K), lambda b, hg: (hg, 0, 0)),
        ],
        out_specs=pl.BlockSpec((1, HB, N, N), lambda b, hg: (b, hg, 0, 0)),
        out_shape=jax.ShapeDtypeStruct((B, H, N, N), jnp.float32),
    )(onehot, m)
    return out
